# baseline (speedup 1.0000x reference)
import functools

import jax
import jax.numpy as jnp
from jax import lax
from jax.experimental import pallas as pl
from jax.experimental.pallas import tpu as pltpu
from jax.experimental.pallas import tpu_sc as plsc

_N = 10000
_E = 320000
_D = 128
_H = 8
_DPH = _D // _H

_NC = 2
_NS = 16
_L = 16
_NW = _NC * _NS

_EPW = _E // _NW
_ECH = 40
_NCH = _EPW // _ECH
_NP = 10240
_RPT = _NP // _NS
_ZR = _ECH
_DEN_W = _L

_RB = 2000


def _ln(h, g, b, eps=1e-5):
    mu = jnp.mean(h, axis=-1, keepdims=True)
    d = h - mu
    var = jnp.mean(d * d, axis=-1, keepdims=True)
    return d * lax.rsqrt(var + eps) * g + b


def _qkv_body(x_ref, wq_ref, wk_ref, wv_ref, q_ref, k_ref, v_ref):
    xb = x_ref[...]
    q_ref[...] = jnp.dot(xb, wq_ref[...],
                         preferred_element_type=jnp.float32) * (1.0 / 4.0)
    k_ref[...] = jnp.dot(xb, wk_ref[...], preferred_element_type=jnp.float32)
    v_ref[...] = jnp.dot(xb, wv_ref[...], preferred_element_type=jnp.float32)


def _qkv(x, wq, wk, wv):
    full = pl.BlockSpec((_D, _D), lambda i: (0, 0))
    row = pl.BlockSpec((_RB, _D), lambda i: (i, 0))
    return pl.pallas_call(
        _qkv_body,
        grid=(_N // _RB,),
        in_specs=[row, full, full, full],
        out_specs=[row, row, row],
        out_shape=[jax.ShapeDtypeStruct((_N, _D), jnp.float32)] * 3,
    )(x, wq, wk, wv)


def _edge_body(q_hbm, k_hbm, v_hbm, src_hbm, dst_hbm, num_out, den_out,
               src_v, dst_v, qr, kr, vr, msg, wmsg, acc_num, acc_den,
               semi, semg, sems):
    c = lax.axis_index("c")
    s = lax.axis_index("s")
    wid = s * _NC + c

    def zb(r, carry):
        for col in range(_D // _L):
            msg[r, pl.ds(col * _L, _L)] = jnp.zeros((_L,), jnp.float32)
        wmsg[r, :] = jnp.zeros((_L,), jnp.float32)
        return carry
    lax.fori_loop(0, _ZR, zb, 0)

    base_row = s * _RPT
    def zcp(r, carry):
        pltpu.sync_copy(msg, acc_num.at[pl.ds(base_row + r * _ZR, _ZR)])
        pltpu.sync_copy(wmsg, acc_den.at[pl.ds(base_row + r * _ZR, _ZR)])
        return carry
    lax.fori_loop(0, _RPT // _ZR, zcp, 0)
    plsc.subcore_barrier()

    lane = lax.iota(jnp.int32, _L)
    perms = [(lane ^ sh).reshape(_L, 1) for sh in (1, 2, 4, 8)]
    _dnums = lax.GatherDimensionNumbers(
        offset_dims=(), collapsed_slice_dims=(0,), start_index_map=(0,))

    def _allsum(p):
        for perm in perms:
            p = p + lax.gather(
                p, perm, _dnums, (1,),
                mode=lax.GatherScatterMode.PROMISE_IN_BOUNDS)
        return p

    ebase = wid * _EPW

    pltpu.sync_copy(src_hbm.at[pl.ds(ebase, _ECH)], src_v.at[0])
    pltpu.sync_copy(dst_hbm.at[pl.ds(ebase, _ECH)], dst_v.at[0])
    pltpu.async_copy(q_hbm.at[dst_v.at[0]], qr, semg)
    pltpu.async_copy(k_hbm.at[src_v.at[0]], kr, semg)
    pltpu.async_copy(v_hbm.at[src_v.at[0]], vr, semg)
    pltpu.async_copy(msg, acc_num.at[dst_v.at[0]], sems, add=True)
    pltpu.async_copy(wmsg, acc_den.at[dst_v.at[0]], sems, add=True)

    def chunk(j, carry):
        for p in (0, 1):
            i = 2 * j + p
            pltpu.make_async_copy(q_hbm.at[dst_v.at[p]], qr, semg).wait()
            pltpu.make_async_copy(k_hbm.at[src_v.at[p]], kr, semg).wait()
            pltpu.make_async_copy(v_hbm.at[src_v.at[p]], vr, semg).wait()
            nbase = ebase + (i + 1) * _ECH
            pltpu.async_copy(src_hbm.at[pl.ds(nbase, _ECH)],
                             src_v.at[1 - p], semi)
            pltpu.async_copy(dst_hbm.at[pl.ds(nbase, _ECH)],
                             dst_v.at[1 - p], semi)
            pltpu.make_async_copy(msg, acc_num.at[dst_v.at[p]], sems).wait()
            pltpu.make_async_copy(wmsg, acc_den.at[dst_v.at[p]], sems).wait()

            def egroup(g, ecarry):
                rows_raw = g * _L + lane
                inb = rows_raw < _ECH
                rows = jnp.minimum(rows_raw, _ECH - 1)
                for h in range(_H):
                    sc0 = jnp.zeros((_L,), jnp.float32)
                    sc1 = jnp.zeros((_L,), jnp.float32)
                    for d in range(_DPH):
                        cvec = jnp.full((_L,), h * _DPH + d, jnp.int32)
                        qv = plsc.load_gather(qr, [rows, cvec])
                        kv = plsc.load_gather(kr, [rows, cvec])
                        if d % 2 == 0:
                            sc0 = sc0 + qv * kv
                        else:
                            sc1 = sc1 + qv * kv
                    wv = jnp.exp(jnp.clip(sc0 + sc1, -5.0, 5.0))
                    plsc.store_scatter(
                        wmsg, [rows, jnp.full((_L,), h, jnp.int32)], wv,
                        mask=inb)
                    for d in range(_DPH):
                        cvec = jnp.full((_L,), h * _DPH + d, jnp.int32)
                        vv = plsc.load_gather(vr, [rows, cvec])
                        plsc.store_scatter(msg, [rows, cvec], vv * wv,
                                           mask=inb)
                return ecarry
            lax.fori_loop(0, (_ECH + _L - 1) // _L, egroup, 0)

            pltpu.async_copy(msg, acc_num.at[dst_v.at[p]], sems, add=True)
            pltpu.async_copy(wmsg, acc_den.at[dst_v.at[p]], sems, add=True)
            pltpu.make_async_copy(src_hbm.at[pl.ds(nbase, _ECH)],
                                  src_v.at[1 - p], semi).wait()
            pltpu.make_async_copy(dst_hbm.at[pl.ds(nbase, _ECH)],
                                  dst_v.at[1 - p], semi).wait()
            pltpu.async_copy(q_hbm.at[dst_v.at[1 - p]], qr, semg)
            pltpu.async_copy(k_hbm.at[src_v.at[1 - p]], kr, semg)
            pltpu.async_copy(v_hbm.at[src_v.at[1 - p]], vr, semg)
        return carry
    lax.fori_loop(0, _NCH // 2, chunk, 0)

    pltpu.make_async_copy(q_hbm.at[dst_v.at[0]], qr, semg).wait()
    pltpu.make_async_copy(k_hbm.at[src_v.at[0]], kr, semg).wait()
    pltpu.make_async_copy(v_hbm.at[src_v.at[0]], vr, semg).wait()
    pltpu.make_async_copy(msg, acc_num.at[dst_v.at[0]], sems).wait()
    pltpu.make_async_copy(wmsg, acc_den.at[dst_v.at[0]], sems).wait()

    plsc.subcore_barrier()
    pltpu.sync_copy(acc_num.at[pl.ds(base_row, _RPT)],
                    num_out.at[c, pl.ds(base_row, _RPT)])
    pltpu.sync_copy(acc_den.at[pl.ds(base_row, _RPT)],
                    den_out.at[c, pl.ds(base_row, _RPT)])


@functools.cache
def _make_edge():
  return pl.kernel(
    _edge_body,
    out_type=[jax.ShapeDtypeStruct((_NC, _NP, _D), jnp.float32),
              jax.ShapeDtypeStruct((_NC, _NP, _DEN_W), jnp.float32)],
    mesh=plsc.VectorSubcoreMesh(core_axis_name="c", subcore_axis_name="s"),
    compiler_params=pltpu.CompilerParams(use_tc_tiling_on_sc=False,
                                         needs_layout_passes=False),
    scratch_types=[
        pltpu.VMEM((2, _ECH), jnp.int32),
        pltpu.VMEM((2, _ECH), jnp.int32),
        pltpu.VMEM((_ECH, _D), jnp.float32),
        pltpu.VMEM((_ECH, _D), jnp.float32),
        pltpu.VMEM((_ECH, _D), jnp.float32),
        pltpu.VMEM((_ECH, _D), jnp.float32),
        pltpu.VMEM((_ECH, _DEN_W), jnp.float32),
        pltpu.VMEM_SHARED((_NP, _D), jnp.float32),
        pltpu.VMEM_SHARED((_NP, _DEN_W), jnp.float32),
        pltpu.SemaphoreType.DMA,
        pltpu.SemaphoreType.DMA,
        pltpu.SemaphoreType.DMA,
    ],
  )


def _fuse_body(num_ref, den_ref, x_ref, wo_ref, bo_ref, wf1_ref, bf1_ref,
               wf2_ref, bf2_ref, g1_ref, b1_ref, g2_ref, b2_ref, out_ref):
    num = num_ref[0] + num_ref[1]
    den = den_ref[0] + den_ref[1]
    den8 = den[:, 0:_H]
    den8 = jnp.where(den8 > 0.0, den8, 1.0)
    inv = 1.0 / den8
    rowi = lax.broadcasted_iota(jnp.int32, (_H, _D), 0)
    coli = lax.broadcasted_iota(jnp.int32, (_H, _D), 1)
    expand = (coli // _DPH == rowi).astype(jnp.float32)
    attn = num * jnp.dot(inv, expand, preferred_element_type=jnp.float32)
    h = (jnp.dot(attn, wo_ref[...], preferred_element_type=jnp.float32)
         + bo_ref[...] + x_ref[...])
    h = _ln(h, g1_ref[...], b1_ref[...])
    f = jnp.maximum(
        jnp.dot(h, wf1_ref[...], preferred_element_type=jnp.float32)
        + bf1_ref[...], 0.0)
    f = (jnp.dot(f, wf2_ref[...], preferred_element_type=jnp.float32)
         + bf2_ref[...])
    out_ref[...] = _ln(h + f, g2_ref[...], b2_ref[...])


def _fuse(num_p, den_p, x, wo, bo, wf1, bf1, wf2, bf2, g1, b1, g2, b2):
    row = pl.BlockSpec((_RB, _D), lambda i: (i, 0))
    return pl.pallas_call(
        _fuse_body,
        grid=(_N // _RB,),
        in_specs=[
            pl.BlockSpec((_NC, _RB, _D), lambda i: (0, i, 0)),
            pl.BlockSpec((_NC, _RB, _DEN_W), lambda i: (0, i, 0)),
            row,
            pl.BlockSpec((_D, _D), lambda i: (0, 0)),
            pl.BlockSpec((1, _D), lambda i: (0, 0)),
            pl.BlockSpec((_D, 2 * _D), lambda i: (0, 0)),
            pl.BlockSpec((1, 2 * _D), lambda i: (0, 0)),
            pl.BlockSpec((2 * _D, _D), lambda i: (0, 0)),
            pl.BlockSpec((1, _D), lambda i: (0, 0)),
            pl.BlockSpec((1, _D), lambda i: (0, 0)),
            pl.BlockSpec((1, _D), lambda i: (0, 0)),
            pl.BlockSpec((1, _D), lambda i: (0, 0)),
            pl.BlockSpec((1, _D), lambda i: (0, 0)),
        ],
        out_specs=row,
        out_shape=jax.ShapeDtypeStruct((_N, _D), jnp.float32),
    )(num_p, den_p, x, wo, bo, wf1, bf1, wf2, bf2, g1, b1, g2, b2)


def kernel(x, edge_index, W_q, W_k, W_v, W_o, b_o, W_f1, b_f1, W_f2, b_f2,
           ln1_g, ln1_b, ln2_g, ln2_b):
    pad = jnp.zeros((64,), edge_index.dtype)
    src = jnp.concatenate([edge_index[0], pad])
    dst = jnp.concatenate([edge_index[1], pad])
    q, k, v = _qkv(x, W_q, W_k, W_v)
    num_p, den_p = _make_edge()(q, k, v, src, dst)
    return _fuse(num_p, den_p, x, W_o, b_o.reshape(1, _D), W_f1,
                 b_f1.reshape(1, 2 * _D), W_f2, b_f2.reshape(1, _D),
                 ln1_g.reshape(1, _D), ln1_b.reshape(1, _D),
                 ln2_g.reshape(1, _D), ln2_b.reshape(1, _D))


# speedup vs baseline: 5.0535x; 5.0535x over previous
import functools

import jax
import jax.numpy as jnp
from jax import lax
from jax.experimental import pallas as pl
from jax.experimental.pallas import tpu as pltpu
from jax.experimental.pallas import tpu_sc as plsc

_N = 10000
_E = 320000
_D = 128
_H = 8
_DPH = _D // _H

_NC = 2
_NS = 16
_L = 16
_NW = _NC * _NS

_EPW = _E // _NW
_ECH = 40
_NCH = _EPW // _ECH
_NP = 10240
_RPT = _NP // _NS
_ZR = _ECH
_DEN_W = _L

_RB = 2000


def _ln(h, g, b, eps=1e-5):
    mu = jnp.mean(h, axis=-1, keepdims=True)
    d = h - mu
    var = jnp.mean(d * d, axis=-1, keepdims=True)
    return d * lax.rsqrt(var + eps) * g + b


def _qkv_body(x_ref, wq_ref, wk_ref, wv_ref, q_ref, k_ref, v_ref):
    xb = x_ref[...]
    q_ref[...] = jnp.dot(xb, wq_ref[...],
                         preferred_element_type=jnp.float32) * (1.0 / 4.0)
    k_ref[...] = jnp.dot(xb, wk_ref[...], preferred_element_type=jnp.float32)
    v_ref[...] = jnp.dot(xb, wv_ref[...], preferred_element_type=jnp.float32)


def _qkv(x, wq, wk, wv):
    full = pl.BlockSpec((_D, _D), lambda i: (0, 0))
    row = pl.BlockSpec((_RB, _D), lambda i: (i, 0))
    return pl.pallas_call(
        _qkv_body,
        grid=(_N // _RB,),
        in_specs=[row, full, full, full],
        out_specs=[row, row, row],
        out_shape=[jax.ShapeDtypeStruct((_N, _D), jnp.float32)] * 3,
    )(x, wq, wk, wv)


def _edge_body(q_hbm, k_hbm, v_hbm, src_hbm, dst_hbm, num_out, den_out,
               src_v, dst_v, qr, kr, vr, msg, wmsg, acc_num, acc_den,
               semi, semg, sems):
    c = lax.axis_index("c")
    s = lax.axis_index("s")
    wid = s * _NC + c

    def zb(r, carry):
        for col in range(_D // _L):
            msg[r, pl.ds(col * _L, _L)] = jnp.zeros((_L,), jnp.float32)
        wmsg[r, :] = jnp.zeros((_L,), jnp.float32)
        return carry
    lax.fori_loop(0, _ZR, zb, 0)

    base_row = s * _RPT
    def zcp(r, carry):
        pltpu.sync_copy(msg, acc_num.at[pl.ds(base_row + r * _ZR, _ZR)])
        pltpu.sync_copy(wmsg, acc_den.at[pl.ds(base_row + r * _ZR, _ZR)])
        return carry
    lax.fori_loop(0, _RPT // _ZR, zcp, 0)
    plsc.subcore_barrier()

    lane = lax.iota(jnp.int32, _L)
    last15 = jnp.full((_L, 1), _L - 1, jnp.int32)
    _dnums = lax.GatherDimensionNumbers(
        offset_dims=(), collapsed_slice_dims=(0,), start_index_map=(0,))

    ebase = wid * _EPW

    pltpu.sync_copy(src_hbm.at[pl.ds(ebase, _ECH)], src_v.at[0])
    pltpu.sync_copy(dst_hbm.at[pl.ds(ebase, _ECH)], dst_v.at[0])
    pltpu.async_copy(q_hbm.at[dst_v.at[0]], qr, semg)
    pltpu.async_copy(k_hbm.at[src_v.at[0]], kr, semg)
    pltpu.async_copy(v_hbm.at[src_v.at[0]], vr, semg)
    pltpu.async_copy(msg, acc_num.at[dst_v.at[0]], sems, add=True)
    pltpu.async_copy(wmsg, acc_den.at[dst_v.at[0]], sems, add=True)

    def chunk(j, carry):
        for p in (0, 1):
            i = 2 * j + p
            pltpu.make_async_copy(q_hbm.at[dst_v.at[p]], qr, semg).wait()
            pltpu.make_async_copy(k_hbm.at[src_v.at[p]], kr, semg).wait()
            pltpu.make_async_copy(v_hbm.at[src_v.at[p]], vr, semg).wait()
            nbase = ebase + (i + 1) * _ECH
            pltpu.async_copy(src_hbm.at[pl.ds(nbase, _ECH)],
                             src_v.at[1 - p], semi)
            pltpu.async_copy(dst_hbm.at[pl.ds(nbase, _ECH)],
                             dst_v.at[1 - p], semi)
            pltpu.make_async_copy(msg, acc_num.at[dst_v.at[p]], sems).wait()
            pltpu.make_async_copy(wmsg, acc_den.at[dst_v.at[p]], sems).wait()

            def edge(e, ecarry):
                den_vec = jnp.zeros((_L,), jnp.float32)
                for h in range(_H):
                    sl = pl.ds(h * _DPH, _DPH)
                    cs = plsc.cumsum(qr[e, sl] * kr[e, sl])
                    sc = lax.gather(
                        cs, last15, _dnums, (1,),
                        mode=lax.GatherScatterMode.PROMISE_IN_BOUNDS)
                    sc = jnp.clip(sc, -5.0, 5.0)
                    wv = jnp.exp(sc)
                    msg[e, sl] = vr[e, sl] * wv
                    den_vec = jnp.where(lane == h, wv, den_vec)
                wmsg[e, :] = den_vec
                return ecarry
            lax.fori_loop(0, _ECH, edge, 0)

            pltpu.async_copy(msg, acc_num.at[dst_v.at[p]], sems, add=True)
            pltpu.async_copy(wmsg, acc_den.at[dst_v.at[p]], sems, add=True)
            pltpu.make_async_copy(src_hbm.at[pl.ds(nbase, _ECH)],
                                  src_v.at[1 - p], semi).wait()
            pltpu.make_async_copy(dst_hbm.at[pl.ds(nbase, _ECH)],
                                  dst_v.at[1 - p], semi).wait()
            pltpu.async_copy(q_hbm.at[dst_v.at[1 - p]], qr, semg)
            pltpu.async_copy(k_hbm.at[src_v.at[1 - p]], kr, semg)
            pltpu.async_copy(v_hbm.at[src_v.at[1 - p]], vr, semg)
        return carry
    lax.fori_loop(0, _NCH // 2, chunk, 0)

    pltpu.make_async_copy(q_hbm.at[dst_v.at[0]], qr, semg).wait()
    pltpu.make_async_copy(k_hbm.at[src_v.at[0]], kr, semg).wait()
    pltpu.make_async_copy(v_hbm.at[src_v.at[0]], vr, semg).wait()
    pltpu.make_async_copy(msg, acc_num.at[dst_v.at[0]], sems).wait()
    pltpu.make_async_copy(wmsg, acc_den.at[dst_v.at[0]], sems).wait()

    plsc.subcore_barrier()
    pltpu.sync_copy(acc_num.at[pl.ds(base_row, _RPT)],
                    num_out.at[c, pl.ds(base_row, _RPT)])
    pltpu.sync_copy(acc_den.at[pl.ds(base_row, _RPT)],
                    den_out.at[c, pl.ds(base_row, _RPT)])


@functools.cache
def _make_edge():
  return pl.kernel(
    _edge_body,
    out_type=[jax.ShapeDtypeStruct((_NC, _NP, _D), jnp.float32),
              jax.ShapeDtypeStruct((_NC, _NP, _DEN_W), jnp.float32)],
    mesh=plsc.VectorSubcoreMesh(core_axis_name="c", subcore_axis_name="s"),
    compiler_params=pltpu.CompilerParams(use_tc_tiling_on_sc=False,
                                         needs_layout_passes=False),
    scratch_types=[
        pltpu.VMEM((2, _ECH), jnp.int32),
        pltpu.VMEM((2, _ECH), jnp.int32),
        pltpu.VMEM((_ECH, _D), jnp.float32),
        pltpu.VMEM((_ECH, _D), jnp.float32),
        pltpu.VMEM((_ECH, _D), jnp.float32),
        pltpu.VMEM((_ECH, _D), jnp.float32),
        pltpu.VMEM((_ECH, _DEN_W), jnp.float32),
        pltpu.VMEM_SHARED((_NP, _D), jnp.float32),
        pltpu.VMEM_SHARED((_NP, _DEN_W), jnp.float32),
        pltpu.SemaphoreType.DMA,
        pltpu.SemaphoreType.DMA,
        pltpu.SemaphoreType.DMA,
    ],
  )


def _fuse_body(num_ref, den_ref, x_ref, wo_ref, bo_ref, wf1_ref, bf1_ref,
               wf2_ref, bf2_ref, g1_ref, b1_ref, g2_ref, b2_ref, out_ref):
    num = num_ref[0] + num_ref[1]
    den = den_ref[0] + den_ref[1]
    den8 = den[:, 0:_H]
    den8 = jnp.where(den8 > 0.0, den8, 1.0)
    inv = 1.0 / den8
    rowi = lax.broadcasted_iota(jnp.int32, (_H, _D), 0)
    coli = lax.broadcasted_iota(jnp.int32, (_H, _D), 1)
    expand = (coli // _DPH == rowi).astype(jnp.float32)
    attn = num * jnp.dot(inv, expand, preferred_element_type=jnp.float32)
    h = (jnp.dot(attn, wo_ref[...], preferred_element_type=jnp.float32)
         + bo_ref[...] + x_ref[...])
    h = _ln(h, g1_ref[...], b1_ref[...])
    f = jnp.maximum(
        jnp.dot(h, wf1_ref[...], preferred_element_type=jnp.float32)
        + bf1_ref[...], 0.0)
    f = (jnp.dot(f, wf2_ref[...], preferred_element_type=jnp.float32)
         + bf2_ref[...])
    out_ref[...] = _ln(h + f, g2_ref[...], b2_ref[...])


def _fuse(num_p, den_p, x, wo, bo, wf1, bf1, wf2, bf2, g1, b1, g2, b2):
    row = pl.BlockSpec((_RB, _D), lambda i: (i, 0))
    return pl.pallas_call(
        _fuse_body,
        grid=(_N // _RB,),
        in_specs=[
            pl.BlockSpec((_NC, _RB, _D), lambda i: (0, i, 0)),
            pl.BlockSpec((_NC, _RB, _DEN_W), lambda i: (0, i, 0)),
            row,
            pl.BlockSpec((_D, _D), lambda i: (0, 0)),
            pl.BlockSpec((1, _D), lambda i: (0, 0)),
            pl.BlockSpec((_D, 2 * _D), lambda i: (0, 0)),
            pl.BlockSpec((1, 2 * _D), lambda i: (0, 0)),
            pl.BlockSpec((2 * _D, _D), lambda i: (0, 0)),
            pl.BlockSpec((1, _D), lambda i: (0, 0)),
            pl.BlockSpec((1, _D), lambda i: (0, 0)),
            pl.BlockSpec((1, _D), lambda i: (0, 0)),
            pl.BlockSpec((1, _D), lambda i: (0, 0)),
            pl.BlockSpec((1, _D), lambda i: (0, 0)),
        ],
        out_specs=row,
        out_shape=jax.ShapeDtypeStruct((_N, _D), jnp.float32),
    )(num_p, den_p, x, wo, bo, wf1, bf1, wf2, bf2, g1, b1, g2, b2)


def kernel(x, edge_index, W_q, W_k, W_v, W_o, b_o, W_f1, b_f1, W_f2, b_f2,
           ln1_g, ln1_b, ln2_g, ln2_b):
    pad = jnp.zeros((64,), edge_index.dtype)
    src = jnp.concatenate([edge_index[0], pad])
    dst = jnp.concatenate([edge_index[1], pad])
    q, k, v = _qkv(x, W_q, W_k, W_v)
    num_p, den_p = _make_edge()(q, k, v, src, dst)
    return _fuse(num_p, den_p, x, W_o, b_o.reshape(1, _D), W_f1,
                 b_f1.reshape(1, 2 * _D), W_f2, b_f2.reshape(1, _D),
                 ln1_g.reshape(1, _D), ln1_b.reshape(1, _D),
                 ln2_g.reshape(1, _D), ln2_b.reshape(1, _D))


# speedup vs baseline: 6.9823x; 1.3817x over previous
import functools

import jax
import jax.numpy as jnp
from jax import lax
from jax.experimental import pallas as pl
from jax.experimental.pallas import tpu as pltpu
from jax.experimental.pallas import tpu_sc as plsc

_N = 10000
_E = 320000
_D = 128
_H = 8
_DPH = _D // _H

_NC = 2
_NS = 16
_L = 16
_NW = _NC * _NS

_EPW = _E // _NW
_ECH = 40
_NCH = _EPW // _ECH
_NP = 10240
_RPT = _NP // _NS
_ZR = _ECH
_DEN_W = _L

_RB = 2000


def _ln(h, g, b, eps=1e-5):
    mu = jnp.mean(h, axis=-1, keepdims=True)
    d = h - mu
    var = jnp.mean(d * d, axis=-1, keepdims=True)
    return d * lax.rsqrt(var + eps) * g + b


def _qkv_body(x_ref, wq_ref, wk_ref, wv_ref, q_ref, k_ref, v_ref):
    xb = x_ref[...]
    q_ref[...] = jnp.dot(xb, wq_ref[...],
                         preferred_element_type=jnp.float32) * (1.0 / 4.0)
    k_ref[...] = jnp.dot(xb, wk_ref[...], preferred_element_type=jnp.float32)
    v_ref[...] = jnp.dot(xb, wv_ref[...], preferred_element_type=jnp.float32)


def _qkv(x, wq, wk, wv):
    full = pl.BlockSpec((_D, _D), lambda i: (0, 0))
    row = pl.BlockSpec((_RB, _D), lambda i: (i, 0))
    return pl.pallas_call(
        _qkv_body,
        grid=(_N // _RB,),
        in_specs=[row, full, full, full],
        out_specs=[row, row, row],
        out_shape=[jax.ShapeDtypeStruct((_N, _D), jnp.float32)] * 3,
    )(x, wq, wk, wv)


def _edge_body(q_hbm, k_hbm, v_hbm, src_hbm, dst_hbm, num_out, den_out,
               src_v, dst_v, qr, kr, vr, msg, wmsg, acc_num, acc_den,
               semi, semg, sems):
    c = lax.axis_index("c")
    s = lax.axis_index("s")
    wid = s * _NC + c

    def zb(r, carry):
        for col in range(_D // _L):
            msg[r, pl.ds(col * _L, _L)] = jnp.zeros((_L,), jnp.float32)
        wmsg[r, :] = jnp.zeros((_L,), jnp.float32)
        return carry
    lax.fori_loop(0, _ZR, zb, 0)

    base_row = s * _RPT
    def zcp(r, carry):
        pltpu.sync_copy(msg, acc_num.at[pl.ds(base_row + r * _ZR, _ZR)])
        pltpu.sync_copy(wmsg, acc_den.at[pl.ds(base_row + r * _ZR, _ZR)])
        return carry
    lax.fori_loop(0, _RPT // _ZR, zcp, 0)
    plsc.subcore_barrier()

    lane = lax.iota(jnp.int32, _L)
    last15 = jnp.full((_L, 1), _L - 1, jnp.int32)
    _dnums = lax.GatherDimensionNumbers(
        offset_dims=(), collapsed_slice_dims=(0,), start_index_map=(0,))

    ebase = wid * _EPW

    pltpu.sync_copy(src_hbm.at[pl.ds(ebase, _ECH)], src_v.at[0])
    pltpu.sync_copy(dst_hbm.at[pl.ds(ebase, _ECH)], dst_v.at[0])
    pltpu.async_copy(q_hbm.at[dst_v.at[0]], qr, semg)
    pltpu.async_copy(k_hbm.at[src_v.at[0]], kr, semg)
    pltpu.async_copy(v_hbm.at[src_v.at[0]], vr, semg)
    pltpu.async_copy(msg, acc_num.at[dst_v.at[0]], sems, add=True)
    pltpu.async_copy(wmsg, acc_den.at[dst_v.at[0]], sems, add=True)

    def chunk(j, carry):
        for p in (0, 1):
            i = 2 * j + p
            pltpu.make_async_copy(q_hbm.at[dst_v.at[p]], qr, semg).wait()
            pltpu.make_async_copy(k_hbm.at[src_v.at[p]], kr, semg).wait()
            pltpu.make_async_copy(v_hbm.at[src_v.at[p]], vr, semg).wait()
            nbase = ebase + (i + 1) * _ECH
            pltpu.async_copy(src_hbm.at[pl.ds(nbase, _ECH)],
                             src_v.at[1 - p], semi)
            pltpu.async_copy(dst_hbm.at[pl.ds(nbase, _ECH)],
                             dst_v.at[1 - p], semi)
            pltpu.make_async_copy(msg, acc_num.at[dst_v.at[p]], sems).wait()
            pltpu.make_async_copy(wmsg, acc_den.at[dst_v.at[p]], sems).wait()

            @functools.partial(plsc.parallel_loop, 0, _ECH, unroll=2)
            def edge(e):
                den_vec = jnp.zeros((_L,), jnp.float32)
                for h in range(_H):
                    sl = pl.ds(h * _DPH, _DPH)
                    cs = plsc.cumsum(qr[e, sl] * kr[e, sl])
                    sc = lax.gather(
                        cs, last15, _dnums, (1,),
                        mode=lax.GatherScatterMode.PROMISE_IN_BOUNDS)
                    sc = jnp.clip(sc, -5.0, 5.0)
                    wv = jnp.exp(sc)
                    msg[e, sl] = vr[e, sl] * wv
                    den_vec = jnp.where(lane == h, wv, den_vec)
                wmsg[e, :] = den_vec

            pltpu.async_copy(msg, acc_num.at[dst_v.at[p]], sems, add=True)
            pltpu.async_copy(wmsg, acc_den.at[dst_v.at[p]], sems, add=True)
            pltpu.make_async_copy(src_hbm.at[pl.ds(nbase, _ECH)],
                                  src_v.at[1 - p], semi).wait()
            pltpu.make_async_copy(dst_hbm.at[pl.ds(nbase, _ECH)],
                                  dst_v.at[1 - p], semi).wait()
            pltpu.async_copy(q_hbm.at[dst_v.at[1 - p]], qr, semg)
            pltpu.async_copy(k_hbm.at[src_v.at[1 - p]], kr, semg)
            pltpu.async_copy(v_hbm.at[src_v.at[1 - p]], vr, semg)
        return carry
    lax.fori_loop(0, _NCH // 2, chunk, 0)

    pltpu.make_async_copy(q_hbm.at[dst_v.at[0]], qr, semg).wait()
    pltpu.make_async_copy(k_hbm.at[src_v.at[0]], kr, semg).wait()
    pltpu.make_async_copy(v_hbm.at[src_v.at[0]], vr, semg).wait()
    pltpu.make_async_copy(msg, acc_num.at[dst_v.at[0]], sems).wait()
    pltpu.make_async_copy(wmsg, acc_den.at[dst_v.at[0]], sems).wait()

    plsc.subcore_barrier()
    pltpu.sync_copy(acc_num.at[pl.ds(base_row, _RPT)],
                    num_out.at[c, pl.ds(base_row, _RPT)])
    pltpu.sync_copy(acc_den.at[pl.ds(base_row, _RPT)],
                    den_out.at[c, pl.ds(base_row, _RPT)])


@functools.cache
def _make_edge():
  return pl.kernel(
    _edge_body,
    out_type=[jax.ShapeDtypeStruct((_NC, _NP, _D), jnp.float32),
              jax.ShapeDtypeStruct((_NC, _NP, _DEN_W), jnp.float32)],
    mesh=plsc.VectorSubcoreMesh(core_axis_name="c", subcore_axis_name="s"),
    compiler_params=pltpu.CompilerParams(use_tc_tiling_on_sc=False,
                                         needs_layout_passes=False),
    scratch_types=[
        pltpu.VMEM((2, _ECH), jnp.int32),
        pltpu.VMEM((2, _ECH), jnp.int32),
        pltpu.VMEM((_ECH, _D), jnp.float32),
        pltpu.VMEM((_ECH, _D), jnp.float32),
        pltpu.VMEM((_ECH, _D), jnp.float32),
        pltpu.VMEM((_ECH, _D), jnp.float32),
        pltpu.VMEM((_ECH, _DEN_W), jnp.float32),
        pltpu.VMEM_SHARED((_NP, _D), jnp.float32),
        pltpu.VMEM_SHARED((_NP, _DEN_W), jnp.float32),
        pltpu.SemaphoreType.DMA,
        pltpu.SemaphoreType.DMA,
        pltpu.SemaphoreType.DMA,
    ],
  )


def _fuse_body(num_ref, den_ref, x_ref, wo_ref, bo_ref, wf1_ref, bf1_ref,
               wf2_ref, bf2_ref, g1_ref, b1_ref, g2_ref, b2_ref, out_ref):
    num = num_ref[0] + num_ref[1]
    den = den_ref[0] + den_ref[1]
    den8 = den[:, 0:_H]
    den8 = jnp.where(den8 > 0.0, den8, 1.0)
    inv = 1.0 / den8
    rowi = lax.broadcasted_iota(jnp.int32, (_H, _D), 0)
    coli = lax.broadcasted_iota(jnp.int32, (_H, _D), 1)
    expand = (coli // _DPH == rowi).astype(jnp.float32)
    attn = num * jnp.dot(inv, expand, preferred_element_type=jnp.float32)
    h = (jnp.dot(attn, wo_ref[...], preferred_element_type=jnp.float32)
         + bo_ref[...] + x_ref[...])
    h = _ln(h, g1_ref[...], b1_ref[...])
    f = jnp.maximum(
        jnp.dot(h, wf1_ref[...], preferred_element_type=jnp.float32)
        + bf1_ref[...], 0.0)
    f = (jnp.dot(f, wf2_ref[...], preferred_element_type=jnp.float32)
         + bf2_ref[...])
    out_ref[...] = _ln(h + f, g2_ref[...], b2_ref[...])


def _fuse(num_p, den_p, x, wo, bo, wf1, bf1, wf2, bf2, g1, b1, g2, b2):
    row = pl.BlockSpec((_RB, _D), lambda i: (i, 0))
    return pl.pallas_call(
        _fuse_body,
        grid=(_N // _RB,),
        in_specs=[
            pl.BlockSpec((_NC, _RB, _D), lambda i: (0, i, 0)),
            pl.BlockSpec((_NC, _RB, _DEN_W), lambda i: (0, i, 0)),
            row,
            pl.BlockSpec((_D, _D), lambda i: (0, 0)),
            pl.BlockSpec((1, _D), lambda i: (0, 0)),
            pl.BlockSpec((_D, 2 * _D), lambda i: (0, 0)),
            pl.BlockSpec((1, 2 * _D), lambda i: (0, 0)),
            pl.BlockSpec((2 * _D, _D), lambda i: (0, 0)),
            pl.BlockSpec((1, _D), lambda i: (0, 0)),
            pl.BlockSpec((1, _D), lambda i: (0, 0)),
            pl.BlockSpec((1, _D), lambda i: (0, 0)),
            pl.BlockSpec((1, _D), lambda i: (0, 0)),
            pl.BlockSpec((1, _D), lambda i: (0, 0)),
        ],
        out_specs=row,
        out_shape=jax.ShapeDtypeStruct((_N, _D), jnp.float32),
    )(num_p, den_p, x, wo, bo, wf1, bf1, wf2, bf2, g1, b1, g2, b2)


def kernel(x, edge_index, W_q, W_k, W_v, W_o, b_o, W_f1, b_f1, W_f2, b_f2,
           ln1_g, ln1_b, ln2_g, ln2_b):
    pad = jnp.zeros((64,), edge_index.dtype)
    src = jnp.concatenate([edge_index[0], pad])
    dst = jnp.concatenate([edge_index[1], pad])
    q, k, v = _qkv(x, W_q, W_k, W_v)
    num_p, den_p = _make_edge()(q, k, v, src, dst)
    return _fuse(num_p, den_p, x, W_o, b_o.reshape(1, _D), W_f1,
                 b_f1.reshape(1, 2 * _D), W_f2, b_f2.reshape(1, _D),
                 ln1_g.reshape(1, _D), ln1_b.reshape(1, _D),
                 ln2_g.reshape(1, _D), ln2_b.reshape(1, _D))
